# baseline (speedup 1.0000x reference)
import functools

import jax
import jax.numpy as jnp
from jax import lax
from jax.experimental import pallas as pl
from jax.experimental.pallas import tpu as pltpu
from jax.experimental.pallas import tpu_sc as plsc

N = 10000
NPAD = 10240
E = 320000
NC = 2
NS = 16
NW = NC * NS
PER_W = E // NW
CH = 128
ITERS = 80
PER_WP = ITERS * CH
B = 8
NB = ITERS // B
STRIPE = NPAD // NS
D2 = 128

R = 2048
GRID = (N + R - 1) // R

_SC_PARAMS = pltpu.CompilerParams(needs_layout_passes=False)


@functools.cache
def _make_sc_segment_sum():
    mesh = plsc.VectorSubcoreMesh(core_axis_name="c", subcore_axis_name="s",
                                  num_cores=NC)

    @functools.partial(
        pl.kernel, mesh=mesh,
        out_type=jax.ShapeDtypeStruct((NC, NPAD, D2), jnp.float32),
        scratch_types=[
            pltpu.VMEM((2, B, CH), jnp.int32),
            pltpu.VMEM((2, B, CH), jnp.int32),
            pltpu.VMEM((2, CH, D2), jnp.float32),
            pltpu.VMEM_SHARED((NPAD, D2), jnp.float32),
            pltpu.SemaphoreType.DMA,
            pltpu.SemaphoreType.DMA,
            pltpu.SemaphoreType.DMA,
            pltpu.SemaphoreType.DMA,
            pltpu.SemaphoreType.DMA,
        ],
        compiler_params=_SC_PARAMS,
    )
    def k(src_hbm, dst_hbm, table_hbm, out_hbm,
          idx_s, idx_d, rows, acc, gsem0, gsem1, ssem0, ssem1, isem):
        c = lax.axis_index("c")
        s = lax.axis_index("s")
        wid = s * NC + c
        gsems = (gsem0, gsem1)
        ssems = (ssem0, ssem1)

        zrow = rows.at[0]
        zero16 = jnp.zeros((16,), jnp.float32)

        def zero_body(r, carry):
            for q in range(D2 // 16):
                zrow[r, pl.ds(q * 16, 16)] = zero16
            return carry

        lax.fori_loop(0, CH, zero_body, None)
        for j in range(STRIPE // CH):
            pltpu.sync_copy(zrow, acc.at[pl.ds(s * STRIPE + j * CH, CH)])
        plsc.subcore_barrier()

        def start_idx(j):
            par = j % 2
            return (
                pltpu.async_copy(src_hbm.at[wid, pl.ds(j * B, B)],
                                 idx_s.at[par], isem),
                pltpu.async_copy(dst_hbm.at[wid, pl.ds(j * B, B)],
                                 idx_d.at[par], isem),
            )

        h0, h1 = start_idx(0)
        h0.wait()
        h1.wait()
        pending = start_idx(1)
        gh = pltpu.async_copy(table_hbm.at[idx_s.at[0, 0]], rows.at[0],
                              gsems[0])
        sh = [None, None]
        for cchunk in range(ITERS):
            j, p = divmod(cchunk, B)
            par = j % 2
            gh.wait()
            nc = cchunk + 1
            if nc < ITERS:
                nj, npp = divmod(nc, B)
                npar = nj % 2
                if npp == 0:
                    pending[0].wait()
                    pending[1].wait()
                if sh[nc % 2] is not None:
                    sh[nc % 2].wait()
                    sh[nc % 2] = None
                ghn = pltpu.async_copy(table_hbm.at[idx_s.at[npar, npp]],
                                       rows.at[nc % 2], gsems[nc % 2])
            sh[cchunk % 2] = pltpu.async_copy(
                rows.at[cchunk % 2], acc.at[idx_d.at[par, p]],
                ssems[cchunk % 2], add=True)
            if nc < ITERS:
                if npp == 0 and nj + 1 < NB:
                    for hbuf in range(2):
                        if sh[hbuf] is not None:
                            sh[hbuf].wait()
                            sh[hbuf] = None
                    pending = start_idx(nj + 1)
                gh = ghn
        for hbuf in range(2):
            if sh[hbuf] is not None:
                sh[hbuf].wait()
        plsc.subcore_barrier()
        pltpu.sync_copy(
            acc.at[pl.ds(s * STRIPE, STRIPE)],
            out_hbm.at[c, pl.ds(s * STRIPE, STRIPE)],
        )

    return k


@functools.cache
def _make_sc_counts():
    mesh = plsc.VectorSubcoreMesh(core_axis_name="c", subcore_axis_name="s",
                                  num_cores=NC)

    @functools.partial(
        pl.kernel, mesh=mesh,
        out_type=jax.ShapeDtypeStruct((NW, NPAD), jnp.float32),
        scratch_types=[
            pltpu.VMEM((ITERS, CH), jnp.int32),
            pltpu.VMEM((NPAD,), jnp.float32),
        ],
        compiler_params=_SC_PARAMS,
    )
    def k(dst_hbm, cnt_hbm, idx_d, cntp):
        c = lax.axis_index("c")
        s = lax.axis_index("s")
        wid = s * NC + c
        pltpu.sync_copy(dst_hbm.at[wid], idx_d)
        zero16 = jnp.zeros((16,), jnp.float32)

        def zero_body(i, carry):
            off = pl.multiple_of(i * 16, 16)
            cntp[pl.ds(off, 16)] = zero16
            return carry

        lax.fori_loop(0, NPAD // 16, zero_body, None)
        ones16 = jnp.ones((16,), jnp.float32)

        def cnt_body(r, carry):
            for b in range(CH // 16):
                d16 = idx_d[r, pl.ds(b * 16, 16)]
                plsc.addupdate_scatter(cntp, [d16], ones16)
            return carry

        lax.fori_loop(0, ITERS, cnt_body, None)
        pltpu.sync_copy(cntp, cnt_hbm.at[wid])

    return k


def _tc1_math(a, cnt32, x, wl, wr, b):
    cnt = lax.dot_general(cnt32, jnp.ones((cnt32.shape[0], 1), jnp.float32),
                          (((0,), (0,)), ((), ())),
                          preferred_element_type=jnp.float32)
    ic = 1.0 / jnp.maximum(cnt, 1.0)
    mean = a * ic
    h = (
        jnp.dot(mean, wl, preferred_element_type=jnp.float32)
        + jnp.dot(x, wr, preferred_element_type=jnp.float32)
        + b
    )
    return jnp.maximum(h, 0.0), jnp.broadcast_to(ic, (ic.shape[0], 8))


def _tc1_body(agg_ref, cnt_ref, x_ref, wl_ref, wr_ref, b_ref, h1_ref, ic_ref):
    h1, ic = _tc1_math(agg_ref[0] + agg_ref[1], cnt_ref[...], x_ref[...],
                       wl_ref[...], wr_ref[...], b_ref[...])
    h1_ref[...] = h1
    ic_ref[...] = ic


def _tc2_math(a, h1, ic, t, dg, wl, wr, b, tfew, tfeb, dew, deb,
              wencw, wencb, wxw, wxb, combw, combb, ow, ob):
    mean = a * ic[:, 0:1]
    h2 = (
        jnp.dot(mean, wl, preferred_element_type=jnp.float32)
        + jnp.dot(h1, wr, preferred_element_type=jnp.float32)
        + b
    )
    tfe = t * tfew + tfeb
    de = dg * dew + deb
    ep0 = jnp.tanh(jnp.dot(tfe, wencw, preferred_element_type=jnp.float32) + wencb)
    ep1 = jnp.tanh(jnp.dot(de, wencw, preferred_element_type=jnp.float32) + wencb)
    xp = jnp.tanh(jnp.dot(h2, wxw, preferred_element_type=jnp.float32) + wxb)
    s0 = jnp.sum(ep0 * xp, axis=1, keepdims=True)
    s1 = jnp.sum(ep1 * xp, axis=1, keepdims=True)
    m = jnp.maximum(s0, s1)
    e0 = jnp.exp(s0 - m)
    e1 = jnp.exp(s1 - m)
    z = e0 + e1
    ctx = tfe * (e0 / z) + de * (e1 / z)
    h1c = (
        jnp.dot(h2, combw[:D2], preferred_element_type=jnp.float32)
        + jnp.dot(ctx, combw[D2:], preferred_element_type=jnp.float32)
        + combb
    )
    logits = jnp.dot(h1c, ow, preferred_element_type=jnp.float32) + ob
    mm = jnp.max(logits, axis=1, keepdims=True)
    lse = jnp.log(jnp.sum(jnp.exp(logits - mm), axis=1, keepdims=True)) + mm
    return h1c, logits - lse


def _tc2_body(agg_ref, h1_ref, ic_ref, t_ref, dg_ref, wl_ref, wr_ref, b_ref,
              tfew_ref, tfeb_ref, dew_ref, deb_ref, wencw_ref, wencb_ref,
              wxw_ref, wxb_ref, combw_ref, combb_ref, ow_ref, ob_ref,
              h1c_ref, out_ref):
    h1c, out = _tc2_math(
        agg_ref[0] + agg_ref[1], h1_ref[...], ic_ref[...], t_ref[...], dg_ref[...],
        wl_ref[...], wr_ref[...], b_ref[...], tfew_ref[...], tfeb_ref[...],
        dew_ref[...], deb_ref[...], wencw_ref[...], wencb_ref[...],
        wxw_ref[...], wxb_ref[...], combw_ref[...], combb_ref[...],
        ow_ref[...], ob_ref[...],
    )
    h1c_ref[...] = h1c
    out_ref[...] = out


def _full(shape):
    return pl.BlockSpec(shape, lambda i: tuple(0 for _ in shape))


def _tc1_call(agg1, cnt32, x, wl, wr, b):
    return pl.pallas_call(
        _tc1_body,
        grid=(GRID,),
        in_specs=[
            pl.BlockSpec((NC, R, D2), lambda i: (0, i, 0)),
            pl.BlockSpec((NW, R), lambda i: (0, i)),
            pl.BlockSpec((R, D2), lambda i: (i, 0)),
            _full((D2, D2)),
            _full((D2, D2)),
            _full((1, D2)),
        ],
        out_specs=[
            pl.BlockSpec((R, D2), lambda i: (i, 0)),
            pl.BlockSpec((R, 8), lambda i: (i, 0)),
        ],
        out_shape=[
            jax.ShapeDtypeStruct((N, D2), jnp.float32),
            jax.ShapeDtypeStruct((N, 8), jnp.float32),
        ],
    )(agg1, cnt32, x, wl, wr, b)


def _tc2_call(agg2, h1, ic, t, dg, wl, wr, b, tfew, tfeb, dew, deb,
              wencw, wencb, wxw, wxb, combw, combb, ow, ob):
    return pl.pallas_call(
        _tc2_body,
        grid=(GRID,),
        in_specs=[
            pl.BlockSpec((NC, R, D2), lambda i: (0, i, 0)),
            pl.BlockSpec((R, D2), lambda i: (i, 0)),
            pl.BlockSpec((R, 8), lambda i: (i, 0)),
            pl.BlockSpec((R, 1), lambda i: (i, 0)),
            pl.BlockSpec((R, 1), lambda i: (i, 0)),
            _full((D2, D2)),
            _full((D2, D2)),
            _full((1, D2)),
            _full((1, 8)),
            _full((1, 8)),
            _full((1, 8)),
            _full((1, 8)),
            _full((8, 8)),
            _full((1, 8)),
            _full((D2, 8)),
            _full((1, 8)),
            _full((D2 + 8, D2)),
            _full((1, D2)),
            _full((D2, 2)),
            _full((1, 2)),
        ],
        out_specs=[
            pl.BlockSpec((R, D2), lambda i: (i, 0)),
            pl.BlockSpec((R, 2), lambda i: (i, 0)),
        ],
        out_shape=[
            jax.ShapeDtypeStruct((N, D2), jnp.float32),
            jax.ShapeDtypeStruct((N, 2), jnp.float32),
        ],
    )(agg2, h1, ic, t, dg, wl, wr, b, tfew, tfeb, dew, deb,
      wencw, wencb, wxw, wxb, combw, combb, ow, ob)


def kernel(x, edge_index, node_mean_out_time_interval, node_out_degree,
           W_l1, b_l1, W_r1, b_r1, W_l2, b_l2, W_r2, b_r2,
           tfe_W, tfe_b, de_W, de_b, w_enc_W, w_enc_b, w_x_W, w_x_b,
           comb_W, comb_b, out_W, out_b):
    pad = PER_WP - PER_W
    src_r = jnp.pad(edge_index[0].reshape(NW, PER_W),
                    ((0, 0), (0, pad))).reshape(NW, ITERS, CH)
    dst_r = jnp.pad(edge_index[1].reshape(NW, PER_W), ((0, 0), (0, pad)),
                    constant_values=NPAD - 1).reshape(NW, ITERS, CH)

    seg_sum = _make_sc_segment_sum()
    cnt32 = _make_sc_counts()(dst_r)
    agg1 = seg_sum(src_r, dst_r, x)
    h1, ic = _tc1_call(agg1, cnt32, x, W_l1, W_r1, (b_l1 + b_r1).reshape(1, D2))

    agg2 = seg_sum(src_r, dst_r, h1)
    h1c, out = _tc2_call(
        agg2, h1, ic,
        node_mean_out_time_interval.reshape(N, 1),
        node_out_degree.reshape(N, 1),
        W_l2, W_r2, (b_l2 + b_r2).reshape(1, D2),
        tfe_W, tfe_b.reshape(1, 8), de_W, de_b.reshape(1, 8),
        w_enc_W, w_enc_b.reshape(1, 8), w_x_W, w_x_b.reshape(1, 8),
        comb_W, comb_b.reshape(1, D2), out_W, out_b.reshape(1, 2),
    )
    return (h1c, out)


# speedup vs baseline: 1.9838x; 1.9838x over previous
import functools

import jax
import jax.numpy as jnp
from jax import lax
from jax.experimental import pallas as pl
from jax.experimental.pallas import tpu as pltpu
from jax.experimental.pallas import tpu_sc as plsc

N = 10000
NPAD = 10240
E = 320000
NC = 2
NS = 16
NW = NC * NS
PER_W = E // NW
CH = 112
ITERS = 90
PER_WP = ITERS * CH
B = 6
NB = ITERS // B
K = 3
STRIPE = NPAD // NS
D2 = 128

R = 2048
GRID = (N + R - 1) // R

_SC_PARAMS = pltpu.CompilerParams(needs_layout_passes=False)


@functools.cache
def _make_sc_segment_sum():
    mesh = plsc.VectorSubcoreMesh(core_axis_name="c", subcore_axis_name="s",
                                  num_cores=NC)

    @functools.partial(
        pl.kernel, mesh=mesh,
        out_type=jax.ShapeDtypeStruct((NC, NPAD, D2), jnp.float32),
        scratch_types=[
            pltpu.VMEM((2, B, CH), jnp.int32),
            pltpu.VMEM((2, B, CH), jnp.int32),
            pltpu.VMEM((K, CH, D2), jnp.float32),
            pltpu.VMEM_SHARED((NPAD, D2), jnp.float32),
            [pltpu.SemaphoreType.DMA] * K,
            [pltpu.SemaphoreType.DMA] * K,
            pltpu.SemaphoreType.DMA,
        ],
        compiler_params=_SC_PARAMS,
    )
    def k(src_hbm, dst_hbm, table_hbm, out_hbm,
          idx_s, idx_d, rows, acc, gsems, ssems, isem):
        c = lax.axis_index("c")
        s = lax.axis_index("s")
        wid = s * NC + c

        zrow = rows.at[0]
        zero16 = jnp.zeros((16,), jnp.float32)

        def zero_body(r, carry):
            for q in range(D2 // 16):
                zrow[r, pl.ds(q * 16, 16)] = zero16
            return carry

        lax.fori_loop(0, 80, zero_body, None)
        zsrc = zrow.at[pl.ds(0, 80)]
        for j in range(STRIPE // 80):
            pltpu.sync_copy(zsrc, acc.at[pl.ds(s * STRIPE + j * 80, 80)])
        plsc.subcore_barrier()

        def start_idx(j):
            par = j % 2
            return (
                pltpu.async_copy(src_hbm.at[wid, j], idx_s.at[par], isem),
                pltpu.async_copy(dst_hbm.at[wid, j], idx_d.at[par], isem),
            )

        def gather(cc):
            j, p = divmod(cc, B)
            return pltpu.async_copy(table_hbm.at[idx_s.at[j % 2, p]],
                                    rows.at[cc % K], gsems[cc % K])

        h0, h1 = start_idx(0)
        h0.wait()
        h1.wait()
        pending = start_idx(1)
        gh = [None] * K
        gh[0] = gather(0)
        gh[1] = gather(1)
        sh = [None] * K
        for cc in range(ITERS):
            j, p = divmod(cc, B)
            gh[cc % K].wait()
            n2 = cc + 2
            if n2 < ITERS:
                j2, p2 = divmod(n2, B)
                if p2 == 0:
                    pending[0].wait()
                    pending[1].wait()
                if sh[n2 % K] is not None:
                    sh[n2 % K].wait()
                    sh[n2 % K] = None
                gh[n2 % K] = gather(n2)
            sh[cc % K] = pltpu.async_copy(
                rows.at[cc % K], acc.at[idx_d.at[j % 2, p]],
                ssems[cc % K], add=True)
            if n2 < ITERS and p2 == 2 and 1 <= j2 < NB - 1:
                for hbuf in range(K):
                    if sh[hbuf] is not None:
                        sh[hbuf].wait()
                        sh[hbuf] = None
                pending = start_idx(j2 + 1)
        for hbuf in range(K):
            if sh[hbuf] is not None:
                sh[hbuf].wait()
        plsc.subcore_barrier()
        pltpu.sync_copy(
            acc.at[pl.ds(s * STRIPE, STRIPE)],
            out_hbm.at[c, pl.ds(s * STRIPE, STRIPE)],
        )

    return k


@functools.cache
def _make_sc_counts():
    mesh = plsc.VectorSubcoreMesh(core_axis_name="c", subcore_axis_name="s",
                                  num_cores=NC)

    @functools.partial(
        pl.kernel, mesh=mesh,
        out_type=jax.ShapeDtypeStruct((NW, NPAD), jnp.float32),
        scratch_types=[
            pltpu.VMEM((ITERS, CH), jnp.int32),
            pltpu.VMEM((NPAD,), jnp.float32),
        ],
        compiler_params=_SC_PARAMS,
    )
    def k(dst_hbm, cnt_hbm, idx_d, cntp):
        c = lax.axis_index("c")
        s = lax.axis_index("s")
        wid = s * NC + c
        pltpu.sync_copy(dst_hbm.at[wid], idx_d)
        zero16 = jnp.zeros((16,), jnp.float32)

        def zero_body(i, carry):
            off = pl.multiple_of(i * 16, 16)
            cntp[pl.ds(off, 16)] = zero16
            return carry

        lax.fori_loop(0, NPAD // 16, zero_body, None)
        ones16 = jnp.ones((16,), jnp.float32)

        def cnt_body(r, carry):
            for b in range(CH // 16):
                d16 = idx_d[r, pl.ds(b * 16, 16)]
                plsc.addupdate_scatter(cntp, [d16], ones16)
            return carry

        lax.fori_loop(0, ITERS, cnt_body, None)
        pltpu.sync_copy(cntp, cnt_hbm.at[wid])

    return k


def _tc1_math(a, cnt32, x, wl, wr, b):
    cnt = lax.dot_general(cnt32, jnp.ones((cnt32.shape[0], 1), jnp.float32),
                          (((0,), (0,)), ((), ())),
                          preferred_element_type=jnp.float32)
    ic = 1.0 / jnp.maximum(cnt, 1.0)
    mean = a * ic
    h = (
        jnp.dot(mean, wl, preferred_element_type=jnp.float32)
        + jnp.dot(x, wr, preferred_element_type=jnp.float32)
        + b
    )
    return jnp.maximum(h, 0.0), jnp.broadcast_to(ic, (ic.shape[0], 8))


def _tc1_body(agg_ref, cnt_ref, x_ref, wl_ref, wr_ref, b_ref, h1_ref, ic_ref):
    h1, ic = _tc1_math(agg_ref[0] + agg_ref[1], cnt_ref[...], x_ref[...],
                       wl_ref[...], wr_ref[...], b_ref[...])
    h1_ref[...] = h1
    ic_ref[...] = ic


def _tc2_math(a, h1, ic, t, dg, wl, wr, b, tfew, tfeb, dew, deb,
              wencw, wencb, wxw, wxb, combw, combb, ow, ob):
    mean = a * ic[:, 0:1]
    h2 = (
        jnp.dot(mean, wl, preferred_element_type=jnp.float32)
        + jnp.dot(h1, wr, preferred_element_type=jnp.float32)
        + b
    )
    tfe = t * tfew + tfeb
    de = dg * dew + deb
    ep0 = jnp.tanh(jnp.dot(tfe, wencw, preferred_element_type=jnp.float32) + wencb)
    ep1 = jnp.tanh(jnp.dot(de, wencw, preferred_element_type=jnp.float32) + wencb)
    xp = jnp.tanh(jnp.dot(h2, wxw, preferred_element_type=jnp.float32) + wxb)
    s0 = jnp.sum(ep0 * xp, axis=1, keepdims=True)
    s1 = jnp.sum(ep1 * xp, axis=1, keepdims=True)
    m = jnp.maximum(s0, s1)
    e0 = jnp.exp(s0 - m)
    e1 = jnp.exp(s1 - m)
    z = e0 + e1
    ctx = tfe * (e0 / z) + de * (e1 / z)
    h1c = (
        jnp.dot(h2, combw[:D2], preferred_element_type=jnp.float32)
        + jnp.dot(ctx, combw[D2:], preferred_element_type=jnp.float32)
        + combb
    )
    logits = jnp.dot(h1c, ow, preferred_element_type=jnp.float32) + ob
    mm = jnp.max(logits, axis=1, keepdims=True)
    lse = jnp.log(jnp.sum(jnp.exp(logits - mm), axis=1, keepdims=True)) + mm
    return h1c, logits - lse


def _tc2_body(agg_ref, h1_ref, ic_ref, t_ref, dg_ref, wl_ref, wr_ref, b_ref,
              tfew_ref, tfeb_ref, dew_ref, deb_ref, wencw_ref, wencb_ref,
              wxw_ref, wxb_ref, combw_ref, combb_ref, ow_ref, ob_ref,
              h1c_ref, out_ref):
    h1c, out = _tc2_math(
        agg_ref[0] + agg_ref[1], h1_ref[...], ic_ref[...], t_ref[...], dg_ref[...],
        wl_ref[...], wr_ref[...], b_ref[...], tfew_ref[...], tfeb_ref[...],
        dew_ref[...], deb_ref[...], wencw_ref[...], wencb_ref[...],
        wxw_ref[...], wxb_ref[...], combw_ref[...], combb_ref[...],
        ow_ref[...], ob_ref[...],
    )
    h1c_ref[...] = h1c
    out_ref[...] = out


def _full(shape):
    return pl.BlockSpec(shape, lambda i: tuple(0 for _ in shape))


def _tc1_call(agg1, cnt32, x, wl, wr, b):
    return pl.pallas_call(
        _tc1_body,
        grid=(GRID,),
        in_specs=[
            pl.BlockSpec((NC, R, D2), lambda i: (0, i, 0)),
            pl.BlockSpec((NW, R), lambda i: (0, i)),
            pl.BlockSpec((R, D2), lambda i: (i, 0)),
            _full((D2, D2)),
            _full((D2, D2)),
            _full((1, D2)),
        ],
        out_specs=[
            pl.BlockSpec((R, D2), lambda i: (i, 0)),
            pl.BlockSpec((R, 8), lambda i: (i, 0)),
        ],
        out_shape=[
            jax.ShapeDtypeStruct((N, D2), jnp.float32),
            jax.ShapeDtypeStruct((N, 8), jnp.float32),
        ],
    )(agg1, cnt32, x, wl, wr, b)


def _tc2_call(agg2, h1, ic, t, dg, wl, wr, b, tfew, tfeb, dew, deb,
              wencw, wencb, wxw, wxb, combw, combb, ow, ob):
    return pl.pallas_call(
        _tc2_body,
        grid=(GRID,),
        in_specs=[
            pl.BlockSpec((NC, R, D2), lambda i: (0, i, 0)),
            pl.BlockSpec((R, D2), lambda i: (i, 0)),
            pl.BlockSpec((R, 8), lambda i: (i, 0)),
            pl.BlockSpec((R, 1), lambda i: (i, 0)),
            pl.BlockSpec((R, 1), lambda i: (i, 0)),
            _full((D2, D2)),
            _full((D2, D2)),
            _full((1, D2)),
            _full((1, 8)),
            _full((1, 8)),
            _full((1, 8)),
            _full((1, 8)),
            _full((8, 8)),
            _full((1, 8)),
            _full((D2, 8)),
            _full((1, 8)),
            _full((D2 + 8, D2)),
            _full((1, D2)),
            _full((D2, 2)),
            _full((1, 2)),
        ],
        out_specs=[
            pl.BlockSpec((R, D2), lambda i: (i, 0)),
            pl.BlockSpec((R, 2), lambda i: (i, 0)),
        ],
        out_shape=[
            jax.ShapeDtypeStruct((N, D2), jnp.float32),
            jax.ShapeDtypeStruct((N, 2), jnp.float32),
        ],
    )(agg2, h1, ic, t, dg, wl, wr, b, tfew, tfeb, dew, deb,
      wencw, wencb, wxw, wxb, combw, combb, ow, ob)


def kernel(x, edge_index, node_mean_out_time_interval, node_out_degree,
           W_l1, b_l1, W_r1, b_r1, W_l2, b_l2, W_r2, b_r2,
           tfe_W, tfe_b, de_W, de_b, w_enc_W, w_enc_b, w_x_W, w_x_b,
           comb_W, comb_b, out_W, out_b):
    pad = PER_WP - PER_W
    src_p = jnp.pad(edge_index[0].reshape(NW, PER_W), ((0, 0), (0, pad)))
    dst_p = jnp.pad(edge_index[1].reshape(NW, PER_W), ((0, 0), (0, pad)),
                    constant_values=NPAD - 1)
    src_r = src_p.reshape(NW, NB, B, CH)
    dst_r = dst_p.reshape(NW, NB, B, CH)

    seg_sum = _make_sc_segment_sum()
    cnt32 = _make_sc_counts()(dst_p.reshape(NW, ITERS, CH))
    agg1 = seg_sum(src_r, dst_r, x)
    h1, ic = _tc1_call(agg1, cnt32, x, W_l1, W_r1, (b_l1 + b_r1).reshape(1, D2))

    agg2 = seg_sum(src_r, dst_r, h1)
    h1c, out = _tc2_call(
        agg2, h1, ic,
        node_mean_out_time_interval.reshape(N, 1),
        node_out_degree.reshape(N, 1),
        W_l2, W_r2, (b_l2 + b_r2).reshape(1, D2),
        tfe_W, tfe_b.reshape(1, 8), de_W, de_b.reshape(1, 8),
        w_enc_W, w_enc_b.reshape(1, 8), w_x_W, w_x_b.reshape(1, 8),
        comb_W, comb_b.reshape(1, D2), out_W, out_b.reshape(1, 2),
    )
    return (h1c, out)


# speedup vs baseline: 1.9868x; 1.0015x over previous
import functools

import jax
import jax.numpy as jnp
from jax import lax
from jax.experimental import pallas as pl
from jax.experimental.pallas import tpu as pltpu
from jax.experimental.pallas import tpu_sc as plsc

N = 10000
NPAD = 10240
E = 320000
NC = 2
NS = 16
NW = NC * NS
PER_W = E // NW
CH = 80
ITERS = 126
PER_WP = ITERS * CH
B = 6
NB = ITERS // B
K = 4
STRIPE = NPAD // NS
D2 = 128

R = 2048
GRID = (N + R - 1) // R

_SC_PARAMS = pltpu.CompilerParams(needs_layout_passes=False)


@functools.cache
def _make_sc_segment_sum():
    mesh = plsc.VectorSubcoreMesh(core_axis_name="c", subcore_axis_name="s",
                                  num_cores=NC)

    @functools.partial(
        pl.kernel, mesh=mesh,
        out_type=jax.ShapeDtypeStruct((NC, NPAD, D2), jnp.float32),
        scratch_types=[
            pltpu.VMEM((2, B, CH), jnp.int32),
            pltpu.VMEM((2, B, CH), jnp.int32),
            pltpu.VMEM((K, CH, D2), jnp.float32),
            pltpu.VMEM_SHARED((NPAD, D2), jnp.float32),
            [pltpu.SemaphoreType.DMA] * K,
            [pltpu.SemaphoreType.DMA] * K,
            pltpu.SemaphoreType.DMA,
        ],
        compiler_params=_SC_PARAMS,
    )
    def k(src_hbm, dst_hbm, table_hbm, out_hbm,
          idx_s, idx_d, rows, acc, gsems, ssems, isem):
        c = lax.axis_index("c")
        s = lax.axis_index("s")
        wid = s * NC + c

        zrow = rows.at[0]
        zero16 = jnp.zeros((16,), jnp.float32)

        def zero_body(r, carry):
            for q in range(D2 // 16):
                zrow[r, pl.ds(q * 16, 16)] = zero16
            return carry

        lax.fori_loop(0, 80, zero_body, None)
        zsrc = zrow.at[pl.ds(0, 80)]
        for j in range(STRIPE // 80):
            pltpu.sync_copy(zsrc, acc.at[pl.ds(s * STRIPE + j * 80, 80)])
        plsc.subcore_barrier()

        def start_idx(j):
            par = j % 2
            return (
                pltpu.async_copy(src_hbm.at[wid, j], idx_s.at[par], isem),
                pltpu.async_copy(dst_hbm.at[wid, j], idx_d.at[par], isem),
            )

        def gather(cc):
            j, p = divmod(cc, B)
            return pltpu.async_copy(table_hbm.at[idx_s.at[j % 2, p]],
                                    rows.at[cc % K], gsems[cc % K])

        h0, h1 = start_idx(0)
        h0.wait()
        h1.wait()
        pending = start_idx(1)
        gh = [None] * K
        for g0 in range(K - 1):
            gh[g0] = gather(g0)
        sh = [None] * K
        for cc in range(ITERS):
            j, p = divmod(cc, B)
            gh[cc % K].wait()
            n2 = cc + K - 1
            if n2 < ITERS:
                j2, p2 = divmod(n2, B)
                if p2 == 0:
                    pending[0].wait()
                    pending[1].wait()
                if sh[n2 % K] is not None:
                    sh[n2 % K].wait()
                    sh[n2 % K] = None
                gh[n2 % K] = gather(n2)
            sh[cc % K] = pltpu.async_copy(
                rows.at[cc % K], acc.at[idx_d.at[j % 2, p]],
                ssems[cc % K], add=True)
            if n2 < ITERS and p2 == K - 1 and 1 <= j2 < NB - 1:
                for hbuf in range(K):
                    if sh[hbuf] is not None:
                        sh[hbuf].wait()
                        sh[hbuf] = None
                pending = start_idx(j2 + 1)
        for hbuf in range(K):
            if sh[hbuf] is not None:
                sh[hbuf].wait()
        plsc.subcore_barrier()
        pltpu.sync_copy(
            acc.at[pl.ds(s * STRIPE, STRIPE)],
            out_hbm.at[c, pl.ds(s * STRIPE, STRIPE)],
        )

    return k


@functools.cache
def _make_sc_counts():
    mesh = plsc.VectorSubcoreMesh(core_axis_name="c", subcore_axis_name="s",
                                  num_cores=NC)

    @functools.partial(
        pl.kernel, mesh=mesh,
        out_type=jax.ShapeDtypeStruct((NW, NPAD), jnp.float32),
        scratch_types=[
            pltpu.VMEM((ITERS, CH), jnp.int32),
            pltpu.VMEM((NPAD,), jnp.float32),
        ],
        compiler_params=_SC_PARAMS,
    )
    def k(dst_hbm, cnt_hbm, idx_d, cntp):
        c = lax.axis_index("c")
        s = lax.axis_index("s")
        wid = s * NC + c
        pltpu.sync_copy(dst_hbm.at[wid], idx_d)
        zero16 = jnp.zeros((16,), jnp.float32)

        def zero_body(i, carry):
            off = pl.multiple_of(i * 16, 16)
            cntp[pl.ds(off, 16)] = zero16
            return carry

        lax.fori_loop(0, NPAD // 16, zero_body, None)
        ones16 = jnp.ones((16,), jnp.float32)

        def cnt_body(r, carry):
            for b in range(CH // 16):
                d16 = idx_d[r, pl.ds(b * 16, 16)]
                plsc.addupdate_scatter(cntp, [d16], ones16)
            return carry

        lax.fori_loop(0, ITERS, cnt_body, None)
        pltpu.sync_copy(cntp, cnt_hbm.at[wid])

    return k


def _tc1_math(a, cnt32, x, wl, wr, b):
    cnt = lax.dot_general(cnt32, jnp.ones((cnt32.shape[0], 1), jnp.float32),
                          (((0,), (0,)), ((), ())),
                          preferred_element_type=jnp.float32)
    ic = 1.0 / jnp.maximum(cnt, 1.0)
    mean = a * ic
    h = (
        jnp.dot(mean, wl, preferred_element_type=jnp.float32)
        + jnp.dot(x, wr, preferred_element_type=jnp.float32)
        + b
    )
    return jnp.maximum(h, 0.0), jnp.broadcast_to(ic, (ic.shape[0], 8))


def _tc1_body(agg_ref, cnt_ref, x_ref, wl_ref, wr_ref, b_ref, h1_ref, ic_ref):
    h1, ic = _tc1_math(agg_ref[0] + agg_ref[1], cnt_ref[...], x_ref[...],
                       wl_ref[...], wr_ref[...], b_ref[...])
    h1_ref[...] = h1
    ic_ref[...] = ic


def _tc2_math(a, h1, ic, t, dg, wl, wr, b, tfew, tfeb, dew, deb,
              wencw, wencb, wxw, wxb, combw, combb, ow, ob):
    mean = a * ic[:, 0:1]
    h2 = (
        jnp.dot(mean, wl, preferred_element_type=jnp.float32)
        + jnp.dot(h1, wr, preferred_element_type=jnp.float32)
        + b
    )
    tfe = t * tfew + tfeb
    de = dg * dew + deb
    ep0 = jnp.tanh(jnp.dot(tfe, wencw, preferred_element_type=jnp.float32) + wencb)
    ep1 = jnp.tanh(jnp.dot(de, wencw, preferred_element_type=jnp.float32) + wencb)
    xp = jnp.tanh(jnp.dot(h2, wxw, preferred_element_type=jnp.float32) + wxb)
    s0 = jnp.sum(ep0 * xp, axis=1, keepdims=True)
    s1 = jnp.sum(ep1 * xp, axis=1, keepdims=True)
    m = jnp.maximum(s0, s1)
    e0 = jnp.exp(s0 - m)
    e1 = jnp.exp(s1 - m)
    z = e0 + e1
    ctx = tfe * (e0 / z) + de * (e1 / z)
    h1c = (
        jnp.dot(h2, combw[:D2], preferred_element_type=jnp.float32)
        + jnp.dot(ctx, combw[D2:], preferred_element_type=jnp.float32)
        + combb
    )
    logits = jnp.dot(h1c, ow, preferred_element_type=jnp.float32) + ob
    mm = jnp.max(logits, axis=1, keepdims=True)
    lse = jnp.log(jnp.sum(jnp.exp(logits - mm), axis=1, keepdims=True)) + mm
    return h1c, logits - lse


def _tc2_body(agg_ref, h1_ref, ic_ref, t_ref, dg_ref, wl_ref, wr_ref, b_ref,
              tfew_ref, tfeb_ref, dew_ref, deb_ref, wencw_ref, wencb_ref,
              wxw_ref, wxb_ref, combw_ref, combb_ref, ow_ref, ob_ref,
              h1c_ref, out_ref):
    h1c, out = _tc2_math(
        agg_ref[0] + agg_ref[1], h1_ref[...], ic_ref[...], t_ref[...], dg_ref[...],
        wl_ref[...], wr_ref[...], b_ref[...], tfew_ref[...], tfeb_ref[...],
        dew_ref[...], deb_ref[...], wencw_ref[...], wencb_ref[...],
        wxw_ref[...], wxb_ref[...], combw_ref[...], combb_ref[...],
        ow_ref[...], ob_ref[...],
    )
    h1c_ref[...] = h1c
    out_ref[...] = out


def _full(shape):
    return pl.BlockSpec(shape, lambda i: tuple(0 for _ in shape))


def _tc1_call(agg1, cnt32, x, wl, wr, b):
    return pl.pallas_call(
        _tc1_body,
        grid=(GRID,),
        in_specs=[
            pl.BlockSpec((NC, R, D2), lambda i: (0, i, 0)),
            pl.BlockSpec((NW, R), lambda i: (0, i)),
            pl.BlockSpec((R, D2), lambda i: (i, 0)),
            _full((D2, D2)),
            _full((D2, D2)),
            _full((1, D2)),
        ],
        out_specs=[
            pl.BlockSpec((R, D2), lambda i: (i, 0)),
            pl.BlockSpec((R, 8), lambda i: (i, 0)),
        ],
        out_shape=[
            jax.ShapeDtypeStruct((N, D2), jnp.float32),
            jax.ShapeDtypeStruct((N, 8), jnp.float32),
        ],
    )(agg1, cnt32, x, wl, wr, b)


def _tc2_call(agg2, h1, ic, t, dg, wl, wr, b, tfew, tfeb, dew, deb,
              wencw, wencb, wxw, wxb, combw, combb, ow, ob):
    return pl.pallas_call(
        _tc2_body,
        grid=(GRID,),
        in_specs=[
            pl.BlockSpec((NC, R, D2), lambda i: (0, i, 0)),
            pl.BlockSpec((R, D2), lambda i: (i, 0)),
            pl.BlockSpec((R, 8), lambda i: (i, 0)),
            pl.BlockSpec((R, 1), lambda i: (i, 0)),
            pl.BlockSpec((R, 1), lambda i: (i, 0)),
            _full((D2, D2)),
            _full((D2, D2)),
            _full((1, D2)),
            _full((1, 8)),
            _full((1, 8)),
            _full((1, 8)),
            _full((1, 8)),
            _full((8, 8)),
            _full((1, 8)),
            _full((D2, 8)),
            _full((1, 8)),
            _full((D2 + 8, D2)),
            _full((1, D2)),
            _full((D2, 2)),
            _full((1, 2)),
        ],
        out_specs=[
            pl.BlockSpec((R, D2), lambda i: (i, 0)),
            pl.BlockSpec((R, 2), lambda i: (i, 0)),
        ],
        out_shape=[
            jax.ShapeDtypeStruct((N, D2), jnp.float32),
            jax.ShapeDtypeStruct((N, 2), jnp.float32),
        ],
    )(agg2, h1, ic, t, dg, wl, wr, b, tfew, tfeb, dew, deb,
      wencw, wencb, wxw, wxb, combw, combb, ow, ob)


def kernel(x, edge_index, node_mean_out_time_interval, node_out_degree,
           W_l1, b_l1, W_r1, b_r1, W_l2, b_l2, W_r2, b_r2,
           tfe_W, tfe_b, de_W, de_b, w_enc_W, w_enc_b, w_x_W, w_x_b,
           comb_W, comb_b, out_W, out_b):
    pad = PER_WP - PER_W
    src_p = jnp.pad(edge_index[0].reshape(NW, PER_W), ((0, 0), (0, pad)))
    dst_p = jnp.pad(edge_index[1].reshape(NW, PER_W), ((0, 0), (0, pad)),
                    constant_values=NPAD - 1)
    src_r = src_p.reshape(NW, NB, B, CH)
    dst_r = dst_p.reshape(NW, NB, B, CH)

    seg_sum = _make_sc_segment_sum()
    cnt32 = _make_sc_counts()(dst_p.reshape(NW, ITERS, CH))
    agg1 = seg_sum(src_r, dst_r, x)
    h1, ic = _tc1_call(agg1, cnt32, x, W_l1, W_r1, (b_l1 + b_r1).reshape(1, D2))

    agg2 = seg_sum(src_r, dst_r, h1)
    h1c, out = _tc2_call(
        agg2, h1, ic,
        node_mean_out_time_interval.reshape(N, 1),
        node_out_degree.reshape(N, 1),
        W_l2, W_r2, (b_l2 + b_r2).reshape(1, D2),
        tfe_W, tfe_b.reshape(1, 8), de_W, de_b.reshape(1, 8),
        w_enc_W, w_enc_b.reshape(1, 8), w_x_W, w_x_b.reshape(1, 8),
        comb_W, comb_b.reshape(1, D2), out_W, out_b.reshape(1, 2),
    )
    return (h1c, out)


# speedup vs baseline: 3.3281x; 1.6751x over previous
import functools

import jax
import jax.numpy as jnp
from jax import lax
from jax.experimental import pallas as pl
from jax.experimental.pallas import tpu as pltpu
from jax.experimental.pallas import tpu_sc as plsc

N = 10000
NPAD = 10240
E = 320000
NC = 2
NS = 16
NW = NC * NS
PER_W = E // NW
CH = 80
ITERS = 125
B = 5
NB = ITERS // B
K = 4
STRIPE = NPAD // NS
D2 = 128

R = 2048
GRID = (N + R - 1) // R

_SC_PARAMS = pltpu.CompilerParams(needs_layout_passes=False)


@functools.cache
def _make_sc_segment_sum():
    mesh = plsc.VectorSubcoreMesh(core_axis_name="c", subcore_axis_name="s",
                                  num_cores=NC)

    @functools.partial(
        pl.kernel, mesh=mesh,
        out_type=jax.ShapeDtypeStruct((NC, NPAD, D2), jnp.float32),
        scratch_types=[
            pltpu.VMEM((2, B, CH), jnp.int32),
            pltpu.VMEM((2, B, CH), jnp.int32),
            pltpu.VMEM((K, CH, D2), jnp.float32),
            pltpu.VMEM_SHARED((NPAD, D2), jnp.float32),
            [pltpu.SemaphoreType.DMA] * K,
            [pltpu.SemaphoreType.DMA] * K,
            pltpu.SemaphoreType.DMA,
        ],
        compiler_params=_SC_PARAMS,
    )
    def k(src_hbm, dst_hbm, table_hbm, out_hbm,
          idx_s, idx_d, rows, acc, gsems, ssems, isem):
        c = lax.axis_index("c")
        s = lax.axis_index("s")
        wid = s * NC + c

        zrow = rows.at[0]
        zero16 = jnp.zeros((16,), jnp.float32)

        def zero_body(r, carry):
            for q in range(D2 // 16):
                zrow[r, pl.ds(q * 16, 16)] = zero16
            return carry

        lax.fori_loop(0, 80, zero_body, None)
        zsrc = zrow.at[pl.ds(0, 80)]
        for j in range(STRIPE // 80):
            pltpu.sync_copy(zsrc, acc.at[pl.ds(s * STRIPE + j * 80, 80)])
        plsc.subcore_barrier()

        def start_idx(j):
            par = j % 2
            return (
                pltpu.async_copy(src_hbm.at[wid, j], idx_s.at[par], isem),
                pltpu.async_copy(dst_hbm.at[wid, j], idx_d.at[par], isem),
            )

        def gather(cc):
            j, p = divmod(cc, B)
            return pltpu.async_copy(table_hbm.at[idx_s.at[j % 2, p]],
                                    rows.at[cc % K], gsems[cc % K])

        h0, h1 = start_idx(0)
        h0.wait()
        h1.wait()
        pending = start_idx(1)
        gh = [None] * K
        for g0 in range(K - 1):
            gh[g0] = gather(g0)
        sh = [None] * K
        for cc in range(ITERS):
            j, p = divmod(cc, B)
            gh[cc % K].wait()
            n2 = cc + K - 1
            if n2 < ITERS:
                j2, p2 = divmod(n2, B)
                if p2 == 0:
                    pending[0].wait()
                    pending[1].wait()
                if sh[n2 % K] is not None:
                    sh[n2 % K].wait()
                    sh[n2 % K] = None
                gh[n2 % K] = gather(n2)
            sh[cc % K] = pltpu.async_copy(
                rows.at[cc % K], acc.at[idx_d.at[j % 2, p]],
                ssems[cc % K], add=True)
            if n2 < ITERS and p2 == K - 1 and 1 <= j2 < NB - 1:
                for hbuf in range(K):
                    if sh[hbuf] is not None:
                        sh[hbuf].wait()
                        sh[hbuf] = None
                pending = start_idx(j2 + 1)
        for hbuf in range(K):
            if sh[hbuf] is not None:
                sh[hbuf].wait()
        plsc.subcore_barrier()
        pltpu.sync_copy(
            acc.at[pl.ds(s * STRIPE, STRIPE)],
            out_hbm.at[c, pl.ds(s * STRIPE, STRIPE)],
        )

    return k


@functools.cache
def _make_sc_counts():
    mesh = plsc.VectorSubcoreMesh(core_axis_name="c", subcore_axis_name="s",
                                  num_cores=NC)

    @functools.partial(
        pl.kernel, mesh=mesh,
        out_type=jax.ShapeDtypeStruct((NW, NPAD), jnp.float32),
        scratch_types=[
            pltpu.VMEM((ITERS, CH), jnp.int32),
            pltpu.VMEM((NPAD,), jnp.float32),
        ],
        compiler_params=_SC_PARAMS,
    )
    def k(dst_hbm, cnt_hbm, idx_d, cntp):
        c = lax.axis_index("c")
        s = lax.axis_index("s")
        wid = s * NC + c
        pltpu.sync_copy(dst_hbm.at[wid], idx_d)
        zero16 = jnp.zeros((16,), jnp.float32)

        def zero_body(i, carry):
            off = pl.multiple_of(i * 16, 16)
            cntp[pl.ds(off, 16)] = zero16
            return carry

        lax.fori_loop(0, NPAD // 16, zero_body, None)
        ones16 = jnp.ones((16,), jnp.float32)

        def cnt_body(r, carry):
            for b in range(CH // 16):
                d16 = idx_d[r, pl.ds(b * 16, 16)]
                plsc.addupdate_scatter(cntp, [d16], ones16)
            return carry

        lax.fori_loop(0, ITERS, cnt_body, None)
        pltpu.sync_copy(cntp, cnt_hbm.at[wid])

    return k


def _tc1_math(a, cnt32, x, wl, wr, b):
    cnt = lax.dot_general(cnt32, jnp.ones((cnt32.shape[0], 1), jnp.float32),
                          (((0,), (0,)), ((), ())),
                          preferred_element_type=jnp.float32)
    ic = 1.0 / jnp.maximum(cnt, 1.0)
    mean = a * ic
    h = (
        jnp.dot(mean, wl, preferred_element_type=jnp.float32)
        + jnp.dot(x, wr, preferred_element_type=jnp.float32)
        + b
    )
    return jnp.maximum(h, 0.0), jnp.broadcast_to(ic, (ic.shape[0], 8))


def _tc1_body(agg_ref, cnt_ref, x_ref, wl_ref, wr_ref, b_ref, h1_ref, ic_ref):
    h1, ic = _tc1_math(agg_ref[0] + agg_ref[1], cnt_ref[...], x_ref[...],
                       wl_ref[...], wr_ref[...], b_ref[...])
    h1_ref[...] = h1
    ic_ref[...] = ic


def _tc2_math(a, h1, ic, t, dg, wl, wr, b, tfew, tfeb, dew, deb,
              wencw, wencb, wxw, wxb, combw, combb, ow, ob):
    mean = a * ic[:, 0:1]
    h2 = (
        jnp.dot(mean, wl, preferred_element_type=jnp.float32)
        + jnp.dot(h1, wr, preferred_element_type=jnp.float32)
        + b
    )
    tfe = t * tfew + tfeb
    de = dg * dew + deb
    ep0 = jnp.tanh(jnp.dot(tfe, wencw, preferred_element_type=jnp.float32) + wencb)
    ep1 = jnp.tanh(jnp.dot(de, wencw, preferred_element_type=jnp.float32) + wencb)
    xp = jnp.tanh(jnp.dot(h2, wxw, preferred_element_type=jnp.float32) + wxb)
    s0 = jnp.sum(ep0 * xp, axis=1, keepdims=True)
    s1 = jnp.sum(ep1 * xp, axis=1, keepdims=True)
    m = jnp.maximum(s0, s1)
    e0 = jnp.exp(s0 - m)
    e1 = jnp.exp(s1 - m)
    z = e0 + e1
    ctx = tfe * (e0 / z) + de * (e1 / z)
    h1c = (
        jnp.dot(h2, combw[:D2], preferred_element_type=jnp.float32)
        + jnp.dot(ctx, combw[D2:], preferred_element_type=jnp.float32)
        + combb
    )
    logits = jnp.dot(h1c, ow, preferred_element_type=jnp.float32) + ob
    mm = jnp.max(logits, axis=1, keepdims=True)
    lse = jnp.log(jnp.sum(jnp.exp(logits - mm), axis=1, keepdims=True)) + mm
    return h1c, logits - lse


def _tc2_body(agg_ref, h1_ref, ic_ref, t_ref, dg_ref, wl_ref, wr_ref, b_ref,
              tfew_ref, tfeb_ref, dew_ref, deb_ref, wencw_ref, wencb_ref,
              wxw_ref, wxb_ref, combw_ref, combb_ref, ow_ref, ob_ref,
              h1c_ref, out_ref):
    h1c, out = _tc2_math(
        agg_ref[0] + agg_ref[1], h1_ref[...], ic_ref[...], t_ref[...], dg_ref[...],
        wl_ref[...], wr_ref[...], b_ref[...], tfew_ref[...], tfeb_ref[...],
        dew_ref[...], deb_ref[...], wencw_ref[...], wencb_ref[...],
        wxw_ref[...], wxb_ref[...], combw_ref[...], combb_ref[...],
        ow_ref[...], ob_ref[...],
    )
    h1c_ref[...] = h1c
    out_ref[...] = out


def _full(shape):
    return pl.BlockSpec(shape, lambda i: tuple(0 for _ in shape))


def _tc1_call(agg1, cnt32, x, wl, wr, b):
    return pl.pallas_call(
        _tc1_body,
        grid=(GRID,),
        in_specs=[
            pl.BlockSpec((NC, R, D2), lambda i: (0, i, 0)),
            pl.BlockSpec((NW, R), lambda i: (0, i)),
            pl.BlockSpec((R, D2), lambda i: (i, 0)),
            _full((D2, D2)),
            _full((D2, D2)),
            _full((1, D2)),
        ],
        out_specs=[
            pl.BlockSpec((R, D2), lambda i: (i, 0)),
            pl.BlockSpec((R, 8), lambda i: (i, 0)),
        ],
        out_shape=[
            jax.ShapeDtypeStruct((N, D2), jnp.float32),
            jax.ShapeDtypeStruct((N, 8), jnp.float32),
        ],
    )(agg1, cnt32, x, wl, wr, b)


def _tc2_call(agg2, h1, ic, t, dg, wl, wr, b, tfew, tfeb, dew, deb,
              wencw, wencb, wxw, wxb, combw, combb, ow, ob):
    return pl.pallas_call(
        _tc2_body,
        grid=(GRID,),
        in_specs=[
            pl.BlockSpec((NC, R, D2), lambda i: (0, i, 0)),
            pl.BlockSpec((R, D2), lambda i: (i, 0)),
            pl.BlockSpec((R, 8), lambda i: (i, 0)),
            pl.BlockSpec((R, 1), lambda i: (i, 0)),
            pl.BlockSpec((R, 1), lambda i: (i, 0)),
            _full((D2, D2)),
            _full((D2, D2)),
            _full((1, D2)),
            _full((1, 8)),
            _full((1, 8)),
            _full((1, 8)),
            _full((1, 8)),
            _full((8, 8)),
            _full((1, 8)),
            _full((D2, 8)),
            _full((1, 8)),
            _full((D2 + 8, D2)),
            _full((1, D2)),
            _full((D2, 2)),
            _full((1, 2)),
        ],
        out_specs=[
            pl.BlockSpec((R, D2), lambda i: (i, 0)),
            pl.BlockSpec((R, 2), lambda i: (i, 0)),
        ],
        out_shape=[
            jax.ShapeDtypeStruct((N, D2), jnp.float32),
            jax.ShapeDtypeStruct((N, 2), jnp.float32),
        ],
    )(agg2, h1, ic, t, dg, wl, wr, b, tfew, tfeb, dew, deb,
      wencw, wencb, wxw, wxb, combw, combb, ow, ob)


def kernel(x, edge_index, node_mean_out_time_interval, node_out_degree,
           W_l1, b_l1, W_r1, b_r1, W_l2, b_l2, W_r2, b_r2,
           tfe_W, tfe_b, de_W, de_b, w_enc_W, w_enc_b, w_x_W, w_x_b,
           comb_W, comb_b, out_W, out_b):
    src_r = edge_index[0].reshape(NW, NB, B, CH)
    dst_r = edge_index[1].reshape(NW, NB, B, CH)

    seg_sum = _make_sc_segment_sum()
    cnt32 = _make_sc_counts()(edge_index[1].reshape(NW, ITERS, CH))
    agg1 = seg_sum(src_r, dst_r, x)
    h1, ic = _tc1_call(agg1, cnt32, x, W_l1, W_r1, (b_l1 + b_r1).reshape(1, D2))

    agg2 = seg_sum(src_r, dst_r, h1)
    h1c, out = _tc2_call(
        agg2, h1, ic,
        node_mean_out_time_interval.reshape(N, 1),
        node_out_degree.reshape(N, 1),
        W_l2, W_r2, (b_l2 + b_r2).reshape(1, D2),
        tfe_W, tfe_b.reshape(1, 8), de_W, de_b.reshape(1, 8),
        w_enc_W, w_enc_b.reshape(1, 8), w_x_W, w_x_b.reshape(1, 8),
        comb_W, comb_b.reshape(1, D2), out_W, out_b.reshape(1, 2),
    )
    return (h1c, out)


# speedup vs baseline: 3.3431x; 1.0045x over previous
import functools

import jax
import jax.numpy as jnp
from jax import lax
from jax.experimental import pallas as pl
from jax.experimental.pallas import tpu as pltpu
from jax.experimental.pallas import tpu_sc as plsc

N = 10000
NPAD = 10240
E = 320000
NC = 2
NS = 16
NW = NC * NS
PER_W = E // NW
CH = 80
ITERS = 125
B = 5
NB = ITERS // B
K = 4
STRIPE = NPAD // NS
D2 = 128

R = 2048
GRID = (N + R - 1) // R

_SC_PARAMS = pltpu.CompilerParams(needs_layout_passes=False)


@functools.cache
def _make_sc_segment_sum():
    mesh = plsc.VectorSubcoreMesh(core_axis_name="c", subcore_axis_name="s",
                                  num_cores=NC)

    @functools.partial(
        pl.kernel, mesh=mesh,
        out_type=jax.ShapeDtypeStruct((NC, NPAD, D2), jnp.float32),
        scratch_types=[
            pltpu.VMEM((2, B, CH), jnp.int32),
            pltpu.VMEM((2, B, CH), jnp.int32),
            pltpu.VMEM((K, CH, D2), jnp.float32),
            pltpu.VMEM_SHARED((NPAD, D2), jnp.float32),
            [pltpu.SemaphoreType.DMA] * K,
            [pltpu.SemaphoreType.DMA] * K,
            pltpu.SemaphoreType.DMA,
        ],
        compiler_params=_SC_PARAMS,
    )
    def k(src_hbm, dst_hbm, table_hbm, out_hbm,
          idx_s, idx_d, rows, acc, gsems, ssems, isem):
        c = lax.axis_index("c")
        s = lax.axis_index("s")
        wid = s * NC + c

        def start_idx(j):
            par = j % 2
            return (
                pltpu.async_copy(src_hbm.at[wid, j], idx_s.at[par], isem),
                pltpu.async_copy(dst_hbm.at[wid, j], idx_d.at[par], isem),
            )

        h0, h1 = start_idx(0)
        pending = start_idx(1)

        zrow = rows.at[0]
        zero16 = jnp.zeros((16,), jnp.float32)

        def zero_body(r, carry):
            for q in range(D2 // 16):
                zrow[r, pl.ds(q * 16, 16)] = zero16
            return carry

        lax.fori_loop(0, 80, zero_body, None)
        zsrc = zrow.at[pl.ds(0, 80)]
        for j in range(STRIPE // 80):
            pltpu.sync_copy(zsrc, acc.at[pl.ds(s * STRIPE + j * 80, 80)])
        plsc.subcore_barrier()

        def gather(cc):
            j, p = divmod(cc, B)
            return pltpu.async_copy(table_hbm.at[idx_s.at[j % 2, p]],
                                    rows.at[cc % K], gsems[cc % K])

        h0.wait()
        h1.wait()
        gh = [None] * K
        for g0 in range(K - 1):
            gh[g0] = gather(g0)
        sh = [None] * K
        for cc in range(ITERS):
            j, p = divmod(cc, B)
            gh[cc % K].wait()
            n2 = cc + K - 1
            if n2 < ITERS:
                j2, p2 = divmod(n2, B)
                if p2 == 0:
                    pending[0].wait()
                    pending[1].wait()
                if sh[n2 % K] is not None:
                    sh[n2 % K].wait()
                    sh[n2 % K] = None
                gh[n2 % K] = gather(n2)
            sh[cc % K] = pltpu.async_copy(
                rows.at[cc % K], acc.at[idx_d.at[j % 2, p]],
                ssems[cc % K], add=True)
            if n2 < ITERS and p2 == K - 1 and 1 <= j2 < NB - 1:
                for hbuf in range(K):
                    if sh[hbuf] is not None:
                        sh[hbuf].wait()
                        sh[hbuf] = None
                pending = start_idx(j2 + 1)
        for hbuf in range(K):
            if sh[hbuf] is not None:
                sh[hbuf].wait()
        plsc.subcore_barrier()
        pltpu.sync_copy(
            acc.at[pl.ds(s * STRIPE, STRIPE)],
            out_hbm.at[c, pl.ds(s * STRIPE, STRIPE)],
        )

    return k


@functools.cache
def _make_sc_counts():
    mesh = plsc.VectorSubcoreMesh(core_axis_name="c", subcore_axis_name="s",
                                  num_cores=NC)

    @functools.partial(
        pl.kernel, mesh=mesh,
        out_type=jax.ShapeDtypeStruct((NW, NPAD), jnp.float32),
        scratch_types=[
            pltpu.VMEM((ITERS, CH), jnp.int32),
            pltpu.VMEM((NPAD,), jnp.float32),
        ],
        compiler_params=_SC_PARAMS,
    )
    def k(dst_hbm, cnt_hbm, idx_d, cntp):
        c = lax.axis_index("c")
        s = lax.axis_index("s")
        wid = s * NC + c
        pltpu.sync_copy(dst_hbm.at[wid], idx_d)
        zero16 = jnp.zeros((16,), jnp.float32)

        def zero_body(i, carry):
            off = pl.multiple_of(i * 16, 16)
            cntp[pl.ds(off, 16)] = zero16
            return carry

        lax.fori_loop(0, NPAD // 16, zero_body, None)
        ones16 = jnp.ones((16,), jnp.float32)

        def cnt_body(r, carry):
            for b in range(CH // 16):
                d16 = idx_d[r, pl.ds(b * 16, 16)]
                plsc.addupdate_scatter(cntp, [d16], ones16)
            return carry

        lax.fori_loop(0, ITERS, cnt_body, None)
        pltpu.sync_copy(cntp, cnt_hbm.at[wid])

    return k


def _tc1_math(a, cnt32, x, wl, wr, b):
    cnt = lax.dot_general(cnt32, jnp.ones((cnt32.shape[0], 1), jnp.float32),
                          (((0,), (0,)), ((), ())),
                          preferred_element_type=jnp.float32)
    ic = 1.0 / jnp.maximum(cnt, 1.0)
    mean = a * ic
    h = (
        jnp.dot(mean, wl, preferred_element_type=jnp.float32)
        + jnp.dot(x, wr, preferred_element_type=jnp.float32)
        + b
    )
    return jnp.maximum(h, 0.0), jnp.broadcast_to(ic, (ic.shape[0], 8))


def _tc1_body(agg_ref, cnt_ref, x_ref, wl_ref, wr_ref, b_ref, h1_ref, ic_ref):
    h1, ic = _tc1_math(agg_ref[0] + agg_ref[1], cnt_ref[...], x_ref[...],
                       wl_ref[...], wr_ref[...], b_ref[...])
    h1_ref[...] = h1
    ic_ref[...] = ic


def _tc2_math(a, h1, ic, t, dg, wl, wr, b, tfew, tfeb, dew, deb,
              wencw, wencb, wxw, wxb, combw, combb, ow, ob):
    mean = a * ic[:, 0:1]
    h2 = (
        jnp.dot(mean, wl, preferred_element_type=jnp.float32)
        + jnp.dot(h1, wr, preferred_element_type=jnp.float32)
        + b
    )
    tfe = t * tfew + tfeb
    de = dg * dew + deb
    ep0 = jnp.tanh(jnp.dot(tfe, wencw, preferred_element_type=jnp.float32) + wencb)
    ep1 = jnp.tanh(jnp.dot(de, wencw, preferred_element_type=jnp.float32) + wencb)
    xp = jnp.tanh(jnp.dot(h2, wxw, preferred_element_type=jnp.float32) + wxb)
    s0 = jnp.sum(ep0 * xp, axis=1, keepdims=True)
    s1 = jnp.sum(ep1 * xp, axis=1, keepdims=True)
    m = jnp.maximum(s0, s1)
    e0 = jnp.exp(s0 - m)
    e1 = jnp.exp(s1 - m)
    z = e0 + e1
    ctx = tfe * (e0 / z) + de * (e1 / z)
    h1c = (
        jnp.dot(h2, combw[:D2], preferred_element_type=jnp.float32)
        + jnp.dot(ctx, combw[D2:], preferred_element_type=jnp.float32)
        + combb
    )
    logits = jnp.dot(h1c, ow, preferred_element_type=jnp.float32) + ob
    mm = jnp.max(logits, axis=1, keepdims=True)
    lse = jnp.log(jnp.sum(jnp.exp(logits - mm), axis=1, keepdims=True)) + mm
    return h1c, logits - lse


def _tc2_body(agg_ref, h1_ref, ic_ref, t_ref, dg_ref, wl_ref, wr_ref, b_ref,
              tfew_ref, tfeb_ref, dew_ref, deb_ref, wencw_ref, wencb_ref,
              wxw_ref, wxb_ref, combw_ref, combb_ref, ow_ref, ob_ref,
              h1c_ref, out_ref):
    h1c, out = _tc2_math(
        agg_ref[0] + agg_ref[1], h1_ref[...], ic_ref[...], t_ref[...], dg_ref[...],
        wl_ref[...], wr_ref[...], b_ref[...], tfew_ref[...], tfeb_ref[...],
        dew_ref[...], deb_ref[...], wencw_ref[...], wencb_ref[...],
        wxw_ref[...], wxb_ref[...], combw_ref[...], combb_ref[...],
        ow_ref[...], ob_ref[...],
    )
    h1c_ref[...] = h1c
    out_ref[...] = out


def _full(shape):
    return pl.BlockSpec(shape, lambda i: tuple(0 for _ in shape))


def _tc1_call(agg1, cnt32, x, wl, wr, b):
    return pl.pallas_call(
        _tc1_body,
        grid=(GRID,),
        in_specs=[
            pl.BlockSpec((NC, R, D2), lambda i: (0, i, 0)),
            pl.BlockSpec((NW, R), lambda i: (0, i)),
            pl.BlockSpec((R, D2), lambda i: (i, 0)),
            _full((D2, D2)),
            _full((D2, D2)),
            _full((1, D2)),
        ],
        out_specs=[
            pl.BlockSpec((R, D2), lambda i: (i, 0)),
            pl.BlockSpec((R, 8), lambda i: (i, 0)),
        ],
        out_shape=[
            jax.ShapeDtypeStruct((N, D2), jnp.float32),
            jax.ShapeDtypeStruct((N, 8), jnp.float32),
        ],
    )(agg1, cnt32, x, wl, wr, b)


def _tc2_call(agg2, h1, ic, t, dg, wl, wr, b, tfew, tfeb, dew, deb,
              wencw, wencb, wxw, wxb, combw, combb, ow, ob):
    return pl.pallas_call(
        _tc2_body,
        grid=(GRID,),
        in_specs=[
            pl.BlockSpec((NC, R, D2), lambda i: (0, i, 0)),
            pl.BlockSpec((R, D2), lambda i: (i, 0)),
            pl.BlockSpec((R, 8), lambda i: (i, 0)),
            pl.BlockSpec((R, 1), lambda i: (i, 0)),
            pl.BlockSpec((R, 1), lambda i: (i, 0)),
            _full((D2, D2)),
            _full((D2, D2)),
            _full((1, D2)),
            _full((1, 8)),
            _full((1, 8)),
            _full((1, 8)),
            _full((1, 8)),
            _full((8, 8)),
            _full((1, 8)),
            _full((D2, 8)),
            _full((1, 8)),
            _full((D2 + 8, D2)),
            _full((1, D2)),
            _full((D2, 2)),
            _full((1, 2)),
        ],
        out_specs=[
            pl.BlockSpec((R, D2), lambda i: (i, 0)),
            pl.BlockSpec((R, 2), lambda i: (i, 0)),
        ],
        out_shape=[
            jax.ShapeDtypeStruct((N, D2), jnp.float32),
            jax.ShapeDtypeStruct((N, 2), jnp.float32),
        ],
    )(agg2, h1, ic, t, dg, wl, wr, b, tfew, tfeb, dew, deb,
      wencw, wencb, wxw, wxb, combw, combb, ow, ob)


def kernel(x, edge_index, node_mean_out_time_interval, node_out_degree,
           W_l1, b_l1, W_r1, b_r1, W_l2, b_l2, W_r2, b_r2,
           tfe_W, tfe_b, de_W, de_b, w_enc_W, w_enc_b, w_x_W, w_x_b,
           comb_W, comb_b, out_W, out_b):
    src_r = edge_index[0].reshape(NW, NB, B, CH)
    dst_r = edge_index[1].reshape(NW, NB, B, CH)

    seg_sum = _make_sc_segment_sum()
    cnt32 = _make_sc_counts()(edge_index[1].reshape(NW, ITERS, CH))
    agg1 = seg_sum(src_r, dst_r, x)
    h1, ic = _tc1_call(agg1, cnt32, x, W_l1, W_r1, (b_l1 + b_r1).reshape(1, D2))

    agg2 = seg_sum(src_r, dst_r, h1)
    h1c, out = _tc2_call(
        agg2, h1, ic,
        node_mean_out_time_interval.reshape(N, 1),
        node_out_degree.reshape(N, 1),
        W_l2, W_r2, (b_l2 + b_r2).reshape(1, D2),
        tfe_W, tfe_b.reshape(1, 8), de_W, de_b.reshape(1, 8),
        w_enc_W, w_enc_b.reshape(1, 8), w_x_W, w_x_b.reshape(1, 8),
        comb_W, comb_b.reshape(1, D2), out_W, out_b.reshape(1, 2),
    )
    return (h1c, out)
